# baseline (speedup 1.0000x reference)
import functools
import math

import jax
import jax.numpy as jnp
from jax import lax
from jax.experimental import pallas as pl
from jax.experimental.pallas import tpu as pltpu
from jax.experimental.pallas import tpu_sc as plsc

N = 10000
E = 160000
NP = 10240
EP = 163840
NC = 4
CL = 128
DH = 512
RB = 2560
GRID = NP // RB
EB = 128
NB = 80
NBH = 40
NST = NB // NBH
BN_SCALE = 1.0 / math.sqrt(1.0 + 1e-5)

_f32 = jnp.float32


def _deg_kernel_body(dst_hbm, zeros_hbm, ones_hbm, out_hbm, hist, idx_v,
                     ones_v, zeros_v):
    cid = lax.axis_index("c")
    sid = lax.axis_index("s")
    wid = cid * 16 + sid
    pltpu.sync_copy(ones_hbm, ones_v)
    pltpu.sync_copy(zeros_hbm, zeros_v)
    pltpu.sync_copy(zeros_v, hist.at[pl.ds(sid * 640, 640)])
    plsc.subcore_barrier()
    pltpu.sync_copy(dst_hbm.at[wid], idx_v)

    def body(j, carry):
        pltpu.sync_copy(ones_v.at[pl.ds(0, 128)], hist.at[idx_v.at[j]],
                        add=True)
        return carry

    lax.fori_loop(0, 40, body, 0)
    plsc.subcore_barrier()
    pltpu.sync_copy(hist.at[pl.ds(sid * 640, 640)],
                    out_hbm.at[cid, pl.ds(sid * 640, 640)])


@functools.cache
def _deg_call():
    return functools.partial(
        pl.kernel,
        out_type=jax.ShapeDtypeStruct((2, NP), _f32),
        mesh=plsc.VectorSubcoreMesh(core_axis_name="c", subcore_axis_name="s"),
        scratch_types=[
            pltpu.VMEM_SHARED((NP,), _f32),
            pltpu.VMEM((40, 128), jnp.int32),
            pltpu.VMEM((640,), _f32),
            pltpu.VMEM((640,), _f32),
        ],
    )(_deg_kernel_body)


def _agg_kernel_body(xws_hbm, src_hbm, dst_hbm, out_hbm, acc, sidx, didx,
                     rows, sem0, sem1):
    cid = lax.axis_index("c")
    sid = lax.axis_index("s")
    rows0 = rows.at[pl.ds(0, EB)]
    rows1 = rows.at[pl.ds(EB, EB)]
    lin = xws_hbm.at[pl.ds(0, EB)]
    def chunk_body(cc, carry):
        c = cid * 2 + cc
        pltpu.sync_copy(xws_hbm.at[pl.ds(c * NP + sid * 640, 640)],
                        acc.at[pl.ds(sid * 640, 640)])
        plsc.subcore_barrier()

        def half_body(h, carry1):
            pltpu.sync_copy(src_hbm.at[c, sid, pl.ds(h * NBH, NBH)], sidx)
            pltpu.sync_copy(dst_hbm.at[sid, pl.ds(h * NBH, NBH)], didx)
            pltpu.async_copy(xws_hbm.at[sidx.at[0]], rows0, sem0)
            pltpu.async_copy(xws_hbm.at[sidx.at[1]], rows1, sem1)

            def outer(jj, carry2):
                j0 = 2 * jj
                pltpu.make_async_copy(lin, rows0, sem0).wait()
                pltpu.sync_copy(rows0, acc.at[didx.at[j0]], add=True)
                pltpu.async_copy(xws_hbm.at[sidx.at[lax.rem(j0 + 2, NBH)]],
                                 rows0, sem0)

                pltpu.make_async_copy(lin, rows1, sem1).wait()
                pltpu.sync_copy(rows1, acc.at[didx.at[j0 + 1]], add=True)
                pltpu.async_copy(xws_hbm.at[sidx.at[lax.rem(j0 + 3, NBH)]],
                                 rows1, sem1)

                return carry2

            lax.fori_loop(0, NBH // 2, outer, 0)
            pltpu.make_async_copy(lin, rows0, sem0).wait()
            pltpu.make_async_copy(lin, rows1, sem1).wait()
            return carry1

        lax.fori_loop(0, NST, half_body, 0)
        plsc.subcore_barrier()
        pltpu.sync_copy(acc.at[pl.ds(sid * 640, 640)],
                        out_hbm.at[c, pl.ds(sid * 640, 640)])
        plsc.subcore_barrier()
        return carry

    lax.fori_loop(0, 2, chunk_body, 0)


@functools.cache
def _agg_call():
    return functools.partial(
        pl.kernel,
        out_type=jax.ShapeDtypeStruct((NC, NP, CL), _f32),
        mesh=plsc.VectorSubcoreMesh(core_axis_name="c", subcore_axis_name="s"),
        scratch_types=[
            pltpu.VMEM_SHARED((NP, CL), _f32),
            pltpu.VMEM((NBH, EB), jnp.int32),
            pltpu.VMEM((NBH, EB), jnp.int32),
            pltpu.VMEM((2 * EB, CL), _f32),
            pltpu.SemaphoreType.DMA,
            pltpu.SemaphoreType.DMA,
        ],
    )(_agg_kernel_body)


def _a0_body(x_ref, p_ref, w_ref, xws_ref, dis_ref):
    p = p_ref[...]
    deg = 1.0 + p[0] + p[1]
    dis = lax.rsqrt(deg)
    xw = jnp.dot(x_ref[...], w_ref[...], preferred_element_type=_f32)
    xws = xw * dis[:, None]
    for cc in range(NC):
        xws_ref[cc] = xws[:, cc * CL:(cc + 1) * CL]
    dis_ref[...] = dis[None, :]


def _a0_call(x, partials, w1):
    return pl.pallas_call(
        _a0_body,
        grid=(GRID,),
        in_specs=[
            pl.BlockSpec((RB, 256), lambda i: (i, 0)),
            pl.BlockSpec((2, RB), lambda i: (0, i)),
            pl.BlockSpec((256, DH), lambda i: (0, 0)),
        ],
        out_specs=[
            pl.BlockSpec((NC, RB, CL), lambda i: (0, i, 0)),
            pl.BlockSpec((1, RB), lambda i: (0, i)),
        ],
        out_shape=[
            jax.ShapeDtypeStruct((NC, NP, CL), _f32),
            jax.ShapeDtypeStruct((1, NP), _f32),
        ],
    )(x, partials, w1)


def _make_mid_body(has_res):
    def body(*refs):
        if has_res:
            (acc_ref, dis_ref, b_ref, g_ref, be_ref, hprev_ref, w_ref,
             h_ref, xws_ref) = refs
        else:
            (acc_ref, dis_ref, b_ref, g_ref, be_ref, w_ref,
             h_ref, xws_ref) = refs
        dis = dis_ref[0]
        parts = []
        for cc in range(NC):
            sl = slice(cc * CL, (cc + 1) * CL)
            t = acc_ref[cc] * dis[:, None] + b_ref[0, sl]
            r = jnp.maximum(t, 0.0)
            hcc = g_ref[0, sl] * (r * BN_SCALE) + be_ref[0, sl]
            if has_res:
                hcc = hcc + hprev_ref[:, sl]
            parts.append(hcc)
        h = jnp.concatenate(parts, axis=1)
        h_ref[...] = h
        xws = jnp.dot(h, w_ref[...], preferred_element_type=_f32)
        xws = xws * dis[:, None]
        for cc in range(NC):
            xws_ref[cc] = xws[:, cc * CL:(cc + 1) * CL]
    return body


def _mid_call(acc, dis, b, g, be, hprev, wnext):
    has_res = hprev is not None
    in_specs = [
        pl.BlockSpec((NC, RB, CL), lambda i: (0, i, 0)),
        pl.BlockSpec((1, RB), lambda i: (0, i)),
        pl.BlockSpec((1, DH), lambda i: (0, 0)),
        pl.BlockSpec((1, DH), lambda i: (0, 0)),
        pl.BlockSpec((1, DH), lambda i: (0, 0)),
    ]
    args = [acc, dis, b, g, be]
    if has_res:
        in_specs.append(pl.BlockSpec((RB, DH), lambda i: (i, 0)))
        args.append(hprev)
    in_specs.append(pl.BlockSpec((DH, DH), lambda i: (0, 0)))
    args.append(wnext)
    return pl.pallas_call(
        _make_mid_body(has_res),
        grid=(GRID,),
        in_specs=in_specs,
        out_specs=[
            pl.BlockSpec((RB, DH), lambda i: (i, 0)),
            pl.BlockSpec((NC, RB, CL), lambda i: (0, i, 0)),
        ],
        out_shape=[
            jax.ShapeDtypeStruct((NP, DH), _f32),
            jax.ShapeDtypeStruct((NC, NP, CL), _f32),
        ],
    )(*args)


def _a3_body(acc_ref, dis_ref, b_ref, g_ref, be_ref, hprev_ref, cw1_ref,
             cb1_ref, cw2_ref, cb2_ref, wsmd_ref, bsmd_ref, w2p_ref,
             b2p_ref, comm_ref, smd_ref):
    dis = dis_ref[0]
    parts = []
    for cc in range(NC):
        sl = slice(cc * CL, (cc + 1) * CL)
        t = acc_ref[cc] * dis[:, None] + b_ref[0, sl]
        r = jnp.maximum(t, 0.0)
        hcc = g_ref[0, sl] * (r * BN_SCALE) + be_ref[0, sl] + hprev_ref[:, sl]
        parts.append(hcc)
    h3 = jnp.concatenate(parts, axis=1)
    c1 = jnp.maximum(
        jnp.dot(h3, cw1_ref[...], preferred_element_type=_f32) + cb1_ref[0],
        0.0)
    comm_ref[...] = (jnp.dot(c1, cw2_ref[...], preferred_element_type=_f32)
                     + cb2_ref[0])
    t = jnp.maximum(
        jnp.dot(h3, wsmd_ref[...], preferred_element_type=_f32) + bsmd_ref[0],
        0.0)
    z = jnp.dot(t, w2p_ref[...], preferred_element_type=_f32) + b2p_ref[0]
    smd_ref[...] = jax.nn.sigmoid(z)


def _a3_call(acc, dis, b, g, be, hprev, cw1, cb1, cw2, cb2, wsmd, bsmd, w2p,
             b2p):
    return pl.pallas_call(
        _a3_body,
        grid=(GRID,),
        in_specs=[
            pl.BlockSpec((NC, RB, CL), lambda i: (0, i, 0)),
            pl.BlockSpec((1, RB), lambda i: (0, i)),
            pl.BlockSpec((1, DH), lambda i: (0, 0)),
            pl.BlockSpec((1, DH), lambda i: (0, 0)),
            pl.BlockSpec((1, DH), lambda i: (0, 0)),
            pl.BlockSpec((RB, DH), lambda i: (i, 0)),
            pl.BlockSpec((DH, DH), lambda i: (0, 0)),
            pl.BlockSpec((1, DH), lambda i: (0, 0)),
            pl.BlockSpec((DH, CL), lambda i: (0, 0)),
            pl.BlockSpec((1, CL), lambda i: (0, 0)),
            pl.BlockSpec((DH, 768), lambda i: (0, 0)),
            pl.BlockSpec((1, 768), lambda i: (0, 0)),
            pl.BlockSpec((768, CL), lambda i: (0, 0)),
            pl.BlockSpec((1, CL), lambda i: (0, 0)),
        ],
        out_specs=[
            pl.BlockSpec((RB, CL), lambda i: (i, 0)),
            pl.BlockSpec((RB, CL), lambda i: (i, 0)),
        ],
        out_shape=[
            jax.ShapeDtypeStruct((NP, CL), _f32),
            jax.ShapeDtypeStruct((NP, CL), _f32),
        ],
    )(acc, dis, b, g, be, hprev, cw1, cb1, cw2, cb2, wsmd, bsmd, w2p, b2p)


def kernel(x, edge_index, W1, b1, W2, b2, W3, b3, g1, be1, g2, be2, g3, be3,
           sW1, sb1, sW2, sb2, mW1, mb1, mW2, mb2, dW1, db1, dW2, db2,
           cW1, cb1, cW2, cb2):
    src = edge_index[0]
    dst = edge_index[1]
    x = jnp.pad(x, ((0, NP - N), (0, 0)))
    npad = EP - E
    pad_ar = jnp.arange(npad, dtype=jnp.int32)
    src_p = jnp.concatenate([src, (pad_ar * 37) % N])
    dst_p = jnp.concatenate([dst, N + (pad_ar % (NP - N))])
    offs = (jnp.arange(NC, dtype=jnp.int32) * NP)[:, None]
    src_all = (src_p[None, :] + offs).reshape(NC, 16, NB, EB)
    dst3 = dst_p.reshape(16, NB, EB)
    dstd = dst_p.reshape(32, 40, 128)

    zeros640 = jnp.zeros((640,), _f32)
    ones640 = jnp.ones((640,), _f32)

    partials = _deg_call()(dstd, zeros640, ones640)

    xws1, dis = _a0_call(x, partials, W1)
    acc1 = _agg_call()(xws1.reshape(NC * NP, CL), src_all, dst3)
    h1, xws2 = _mid_call(acc1, dis, b1.reshape(1, DH), g1.reshape(1, DH),
                         be1.reshape(1, DH), None, W2)
    acc2 = _agg_call()(xws2.reshape(NC * NP, CL), src_all, dst3)
    h2, xws3 = _mid_call(acc2, dis, b2.reshape(1, DH), g2.reshape(1, DH),
                         be2.reshape(1, DH), h1, W3)
    acc3 = _agg_call()(xws3.reshape(NC * NP, CL), src_all, dst3)

    wsmd = jnp.concatenate([sW1, mW1, dW1], axis=1)
    bsmd = jnp.concatenate([sb1, mb1, db1]).reshape(1, 768)
    w2p = jnp.zeros((768, CL), _f32)
    w2p = w2p.at[0:256, 0].set(sW2[:, 0])
    w2p = w2p.at[256:512, 1].set(mW2[:, 0])
    w2p = w2p.at[512:768, 2].set(dW2[:, 0])
    b2p = jnp.zeros((1, CL), _f32)
    b2p = b2p.at[0, 0].set(sb2[0]).at[0, 1].set(mb2[0]).at[0, 2].set(db2[0])

    communities, smd = _a3_call(
        acc3, dis, b3.reshape(1, DH), g3.reshape(1, DH), be3.reshape(1, DH),
        h2, cW1, cb1.reshape(1, DH), cW2, cb2.reshape(1, CL),
        wsmd, bsmd, w2p, b2p)

    splits = smd[:N, 0:1]
    merges = smd[:N, 1:2]
    dissolves = smd[:N, 2:3]
    return (communities[:N], splits, merges, dissolves)


# speedup vs baseline: 1.0038x; 1.0038x over previous
import functools
import math

import jax
import jax.numpy as jnp
from jax import lax
from jax.experimental import pallas as pl
from jax.experimental.pallas import tpu as pltpu
from jax.experimental.pallas import tpu_sc as plsc

N = 10000
E = 160000
NP = 10240
EP = 163840
NC = 4
CL = 128
DH = 512
RB = 2048
GRID = NP // RB
EB = 128
NB = 80
NBH = 40
NST = NB // NBH
BN_SCALE = 1.0 / math.sqrt(1.0 + 1e-5)

_f32 = jnp.float32


def _deg_kernel_body(dst_hbm, zeros_hbm, ones_hbm, out_hbm, hist, idx_v,
                     ones_v, zeros_v):
    cid = lax.axis_index("c")
    sid = lax.axis_index("s")
    wid = cid * 16 + sid
    pltpu.sync_copy(ones_hbm, ones_v)
    pltpu.sync_copy(zeros_hbm, zeros_v)
    pltpu.sync_copy(zeros_v, hist.at[pl.ds(sid * 640, 640)])
    plsc.subcore_barrier()
    pltpu.sync_copy(dst_hbm.at[wid], idx_v)

    def body(j, carry):
        pltpu.sync_copy(ones_v.at[pl.ds(0, 128)], hist.at[idx_v.at[j]],
                        add=True)
        return carry

    lax.fori_loop(0, 40, body, 0)
    plsc.subcore_barrier()
    pltpu.sync_copy(hist.at[pl.ds(sid * 640, 640)],
                    out_hbm.at[cid, pl.ds(sid * 640, 640)])


@functools.cache
def _deg_call():
    return functools.partial(
        pl.kernel,
        out_type=jax.ShapeDtypeStruct((2, NP), _f32),
        mesh=plsc.VectorSubcoreMesh(core_axis_name="c", subcore_axis_name="s"),
        scratch_types=[
            pltpu.VMEM_SHARED((NP,), _f32),
            pltpu.VMEM((40, 128), jnp.int32),
            pltpu.VMEM((640,), _f32),
            pltpu.VMEM((640,), _f32),
        ],
    )(_deg_kernel_body)


def _agg_kernel_body(xws_hbm, src_hbm, dst_hbm, out_hbm, acc, sidx, didx,
                     rows, sem0, sem1):
    cid = lax.axis_index("c")
    sid = lax.axis_index("s")
    rows0 = rows.at[pl.ds(0, EB)]
    rows1 = rows.at[pl.ds(EB, EB)]
    lin = xws_hbm.at[pl.ds(0, EB)]
    def chunk_body(cc, carry):
        c = cid * 2 + cc
        pltpu.sync_copy(xws_hbm.at[pl.ds(c * NP + sid * 640, 640)],
                        acc.at[pl.ds(sid * 640, 640)])
        plsc.subcore_barrier()

        def half_body(h, carry1):
            pltpu.sync_copy(src_hbm.at[c, sid, pl.ds(h * NBH, NBH)], sidx)
            pltpu.sync_copy(dst_hbm.at[sid, pl.ds(h * NBH, NBH)], didx)
            pltpu.async_copy(xws_hbm.at[sidx.at[0]], rows0, sem0)
            pltpu.async_copy(xws_hbm.at[sidx.at[1]], rows1, sem1)

            def outer(jj, carry2):
                j0 = 2 * jj
                pltpu.make_async_copy(lin, rows0, sem0).wait()
                pltpu.sync_copy(rows0, acc.at[didx.at[j0]], add=True)
                pltpu.async_copy(xws_hbm.at[sidx.at[lax.rem(j0 + 2, NBH)]],
                                 rows0, sem0)

                pltpu.make_async_copy(lin, rows1, sem1).wait()
                pltpu.sync_copy(rows1, acc.at[didx.at[j0 + 1]], add=True)
                pltpu.async_copy(xws_hbm.at[sidx.at[lax.rem(j0 + 3, NBH)]],
                                 rows1, sem1)

                return carry2

            lax.fori_loop(0, NBH // 2, outer, 0)
            pltpu.make_async_copy(lin, rows0, sem0).wait()
            pltpu.make_async_copy(lin, rows1, sem1).wait()
            return carry1

        lax.fori_loop(0, NST, half_body, 0)
        plsc.subcore_barrier()
        pltpu.sync_copy(acc.at[pl.ds(sid * 640, 640)],
                        out_hbm.at[c, pl.ds(sid * 640, 640)])
        plsc.subcore_barrier()
        return carry

    lax.fori_loop(0, 2, chunk_body, 0)


@functools.cache
def _agg_call():
    return functools.partial(
        pl.kernel,
        out_type=jax.ShapeDtypeStruct((NC, NP, CL), _f32),
        mesh=plsc.VectorSubcoreMesh(core_axis_name="c", subcore_axis_name="s"),
        scratch_types=[
            pltpu.VMEM_SHARED((NP, CL), _f32),
            pltpu.VMEM((NBH, EB), jnp.int32),
            pltpu.VMEM((NBH, EB), jnp.int32),
            pltpu.VMEM((2 * EB, CL), _f32),
            pltpu.SemaphoreType.DMA,
            pltpu.SemaphoreType.DMA,
        ],
    )(_agg_kernel_body)


def _a0_body(x_ref, p_ref, w_ref, xws_ref, dis_ref):
    p = p_ref[...]
    deg = 1.0 + p[0] + p[1]
    dis = lax.rsqrt(deg)
    xw = jnp.dot(x_ref[...], w_ref[...], preferred_element_type=_f32)
    xws = xw * dis[:, None]
    for cc in range(NC):
        xws_ref[cc] = xws[:, cc * CL:(cc + 1) * CL]
    dis_ref[...] = dis[None, :]


def _a0_call(x, partials, w1):
    return pl.pallas_call(
        _a0_body,
        grid=(GRID,),
        in_specs=[
            pl.BlockSpec((RB, 256), lambda i: (i, 0)),
            pl.BlockSpec((2, RB), lambda i: (0, i)),
            pl.BlockSpec((256, DH), lambda i: (0, 0)),
        ],
        out_specs=[
            pl.BlockSpec((NC, RB, CL), lambda i: (0, i, 0)),
            pl.BlockSpec((1, RB), lambda i: (0, i)),
        ],
        out_shape=[
            jax.ShapeDtypeStruct((NC, NP, CL), _f32),
            jax.ShapeDtypeStruct((1, NP), _f32),
        ],
    )(x, partials, w1)


def _make_mid_body(has_res):
    def body(*refs):
        if has_res:
            (acc_ref, dis_ref, b_ref, g_ref, be_ref, hprev_ref, w_ref,
             h_ref, xws_ref) = refs
        else:
            (acc_ref, dis_ref, b_ref, g_ref, be_ref, w_ref,
             h_ref, xws_ref) = refs
        dis = dis_ref[0]
        parts = []
        for cc in range(NC):
            sl = slice(cc * CL, (cc + 1) * CL)
            t = acc_ref[cc] * dis[:, None] + b_ref[0, sl]
            r = jnp.maximum(t, 0.0)
            hcc = g_ref[0, sl] * (r * BN_SCALE) + be_ref[0, sl]
            if has_res:
                hcc = hcc + hprev_ref[:, sl]
            parts.append(hcc)
        h = jnp.concatenate(parts, axis=1)
        h_ref[...] = h
        xws = jnp.dot(h, w_ref[...], preferred_element_type=_f32)
        xws = xws * dis[:, None]
        for cc in range(NC):
            xws_ref[cc] = xws[:, cc * CL:(cc + 1) * CL]
    return body


def _mid_call(acc, dis, b, g, be, hprev, wnext):
    has_res = hprev is not None
    in_specs = [
        pl.BlockSpec((NC, RB, CL), lambda i: (0, i, 0)),
        pl.BlockSpec((1, RB), lambda i: (0, i)),
        pl.BlockSpec((1, DH), lambda i: (0, 0)),
        pl.BlockSpec((1, DH), lambda i: (0, 0)),
        pl.BlockSpec((1, DH), lambda i: (0, 0)),
    ]
    args = [acc, dis, b, g, be]
    if has_res:
        in_specs.append(pl.BlockSpec((RB, DH), lambda i: (i, 0)))
        args.append(hprev)
    in_specs.append(pl.BlockSpec((DH, DH), lambda i: (0, 0)))
    args.append(wnext)
    return pl.pallas_call(
        _make_mid_body(has_res),
        grid=(GRID,),
        in_specs=in_specs,
        out_specs=[
            pl.BlockSpec((RB, DH), lambda i: (i, 0)),
            pl.BlockSpec((NC, RB, CL), lambda i: (0, i, 0)),
        ],
        out_shape=[
            jax.ShapeDtypeStruct((NP, DH), _f32),
            jax.ShapeDtypeStruct((NC, NP, CL), _f32),
        ],
    )(*args)


def _a3_body(acc_ref, dis_ref, b_ref, g_ref, be_ref, hprev_ref, cw1_ref,
             cb1_ref, cw2_ref, cb2_ref, wsmd_ref, bsmd_ref, w2p_ref,
             b2p_ref, comm_ref, smd_ref):
    dis = dis_ref[0]
    parts = []
    for cc in range(NC):
        sl = slice(cc * CL, (cc + 1) * CL)
        t = acc_ref[cc] * dis[:, None] + b_ref[0, sl]
        r = jnp.maximum(t, 0.0)
        hcc = g_ref[0, sl] * (r * BN_SCALE) + be_ref[0, sl] + hprev_ref[:, sl]
        parts.append(hcc)
    h3 = jnp.concatenate(parts, axis=1)
    c1 = jnp.maximum(
        jnp.dot(h3, cw1_ref[...], preferred_element_type=_f32) + cb1_ref[0],
        0.0)
    comm_ref[...] = (jnp.dot(c1, cw2_ref[...], preferred_element_type=_f32)
                     + cb2_ref[0])
    t = jnp.maximum(
        jnp.dot(h3, wsmd_ref[...], preferred_element_type=_f32) + bsmd_ref[0],
        0.0)
    z = jnp.dot(t, w2p_ref[...], preferred_element_type=_f32) + b2p_ref[0]
    smd_ref[...] = jax.nn.sigmoid(z)


def _a3_call(acc, dis, b, g, be, hprev, cw1, cb1, cw2, cb2, wsmd, bsmd, w2p,
             b2p):
    return pl.pallas_call(
        _a3_body,
        grid=(GRID,),
        in_specs=[
            pl.BlockSpec((NC, RB, CL), lambda i: (0, i, 0)),
            pl.BlockSpec((1, RB), lambda i: (0, i)),
            pl.BlockSpec((1, DH), lambda i: (0, 0)),
            pl.BlockSpec((1, DH), lambda i: (0, 0)),
            pl.BlockSpec((1, DH), lambda i: (0, 0)),
            pl.BlockSpec((RB, DH), lambda i: (i, 0)),
            pl.BlockSpec((DH, DH), lambda i: (0, 0)),
            pl.BlockSpec((1, DH), lambda i: (0, 0)),
            pl.BlockSpec((DH, CL), lambda i: (0, 0)),
            pl.BlockSpec((1, CL), lambda i: (0, 0)),
            pl.BlockSpec((DH, 768), lambda i: (0, 0)),
            pl.BlockSpec((1, 768), lambda i: (0, 0)),
            pl.BlockSpec((768, CL), lambda i: (0, 0)),
            pl.BlockSpec((1, CL), lambda i: (0, 0)),
        ],
        out_specs=[
            pl.BlockSpec((RB, CL), lambda i: (i, 0)),
            pl.BlockSpec((RB, CL), lambda i: (i, 0)),
        ],
        out_shape=[
            jax.ShapeDtypeStruct((NP, CL), _f32),
            jax.ShapeDtypeStruct((NP, CL), _f32),
        ],
    )(acc, dis, b, g, be, hprev, cw1, cb1, cw2, cb2, wsmd, bsmd, w2p, b2p)


def kernel(x, edge_index, W1, b1, W2, b2, W3, b3, g1, be1, g2, be2, g3, be3,
           sW1, sb1, sW2, sb2, mW1, mb1, mW2, mb2, dW1, db1, dW2, db2,
           cW1, cb1, cW2, cb2):
    src = edge_index[0]
    dst = edge_index[1]
    x = jnp.pad(x, ((0, NP - N), (0, 0)))
    npad = EP - E
    pad_ar = jnp.arange(npad, dtype=jnp.int32)
    src_p = jnp.concatenate([src, (pad_ar * 37) % N])
    dst_p = jnp.concatenate([dst, N + (pad_ar % (NP - N))])
    offs = (jnp.arange(NC, dtype=jnp.int32) * NP)[:, None]
    src_all = (src_p[None, :] + offs).reshape(NC, 16, NB, EB)
    dst3 = dst_p.reshape(16, NB, EB)
    dstd = dst_p.reshape(32, 40, 128)

    zeros640 = jnp.zeros((640,), _f32)
    ones640 = jnp.ones((640,), _f32)

    partials = _deg_call()(dstd, zeros640, ones640)

    xws1, dis = _a0_call(x, partials, W1)
    acc1 = _agg_call()(xws1.reshape(NC * NP, CL), src_all, dst3)
    h1, xws2 = _mid_call(acc1, dis, b1.reshape(1, DH), g1.reshape(1, DH),
                         be1.reshape(1, DH), None, W2)
    acc2 = _agg_call()(xws2.reshape(NC * NP, CL), src_all, dst3)
    h2, xws3 = _mid_call(acc2, dis, b2.reshape(1, DH), g2.reshape(1, DH),
                         be2.reshape(1, DH), h1, W3)
    acc3 = _agg_call()(xws3.reshape(NC * NP, CL), src_all, dst3)

    wsmd = jnp.concatenate([sW1, mW1, dW1], axis=1)
    bsmd = jnp.concatenate([sb1, mb1, db1]).reshape(1, 768)
    w2p = jnp.zeros((768, CL), _f32)
    w2p = w2p.at[0:256, 0].set(sW2[:, 0])
    w2p = w2p.at[256:512, 1].set(mW2[:, 0])
    w2p = w2p.at[512:768, 2].set(dW2[:, 0])
    b2p = jnp.zeros((1, CL), _f32)
    b2p = b2p.at[0, 0].set(sb2[0]).at[0, 1].set(mb2[0]).at[0, 2].set(db2[0])

    communities, smd = _a3_call(
        acc3, dis, b3.reshape(1, DH), g3.reshape(1, DH), be3.reshape(1, DH),
        h2, cW1, cb1.reshape(1, DH), cW2, cb2.reshape(1, CL),
        wsmd, bsmd, w2p, b2p)

    splits = smd[:N, 0:1]
    merges = smd[:N, 1:2]
    dissolves = smd[:N, 2:3]
    return (communities[:N], splits, merges, dissolves)
